# baseline (speedup 1.0000x reference)
import numpy as np

import jax
import jax.numpy as jnp
from jax import lax
from jax.experimental import pallas as pl
from jax.experimental.pallas import tpu as pltpu

ROWS = 64
COLS = 1_000_000
SUB = 8
LANES = COLS // SUB
EPS = np.float32(1e-10)
BIG = np.int32(2**30)

_K0 = np.uint32(0)
_K1 = np.uint32(42)
_K2 = np.uint32(0x1BD11BDA ^ 42)
_ROT = (13, 15, 26, 6, 17, 29, 16, 24)


def _rotl(x, r):
    return lax.shift_left(x, np.uint32(r)) | lax.shift_right_logical(
        x, np.uint32(32 - r)
    )


def _threefry_bits(c1):
    x0 = jnp.zeros_like(c1)
    x1 = c1 + _K1
    inject = ((_K1, _K2), (_K2, _K0), (_K0, _K1), (_K1, _K2), (_K2, _K0))
    for g in range(5):
        rots = _ROT[:4] if g % 2 == 0 else _ROT[4:]
        for r in rots:
            x0 = x0 + x1
            x1 = _rotl(x1, r)
            x1 = x1 ^ x0
        a, b = inject[g]
        x0 = x0 + a
        x1 = x1 + b + np.uint32(g + 1)
    return x0 ^ x1


def _row_kernel(temps_ref, logits_ref, out_ref):
    r = pl.program_id(0)
    x = logits_ref[...]
    t_raw = temps_ref[r]
    t = jnp.where(t_raw == 0.0, np.float32(1.0), t_raw)

    col = lax.broadcasted_iota(jnp.int32, (SUB, LANES), 1)
    row8 = lax.broadcasted_iota(jnp.int32, (SUB, LANES), 0)
    flat = row8 * LANES + col

    vmax = jnp.max(x)
    greedy = jnp.min(jnp.where(x == vmax, flat, BIG))

    c1 = flat.astype(jnp.uint32) + r.astype(jnp.uint32) * np.uint32(COLS)
    bits = _threefry_bits(c1)
    mant = lax.shift_right_logical(bits, np.uint32(9)) | np.uint32(0x3F800000)
    u = lax.bitcast_convert_type(mant, jnp.float32) - np.float32(1.0)
    expo = -jnp.log1p(-u)

    s = (x / t - vmax / t) - jnp.log(expo + EPS)
    smax = jnp.max(s)
    sample = jnp.min(jnp.where(s == smax, flat, BIG))

    out_ref[r] = jnp.where(t_raw == 0.0, greedy, sample)


@jax.jit
def kernel(logits, temperatures):
    x = logits.reshape(ROWS * SUB, LANES)
    return pl.pallas_call(
        _row_kernel,
        grid=(ROWS,),
        in_specs=[
            pl.BlockSpec(memory_space=pltpu.SMEM),
            pl.BlockSpec((SUB, LANES), lambda r: (r, 0)),
        ],
        out_specs=pl.BlockSpec(memory_space=pltpu.SMEM),
        out_shape=jax.ShapeDtypeStruct((ROWS,), jnp.int32),
    )(temperatures, x)


# speedup vs baseline: 1.4083x; 1.4083x over previous
import numpy as np

import jax
import jax.numpy as jnp
from jax import lax
from jax.experimental import pallas as pl
from jax.experimental.pallas import tpu as pltpu

ROWS = 64
COLS = 1_000_000
SUB = 8
LANES = COLS // SUB
W = 1024
NCHUNK = LANES // W
TAIL = LANES - NCHUNK * W
EPS = np.float32(1e-10)
BIG = np.int32(2**30)
NEG_INF = np.float32("-inf")

_K0 = np.uint32(0)
_K1 = np.uint32(42)
_K2 = np.uint32(0x1BD11BDA ^ 42)
_ROT = (13, 15, 26, 6, 17, 29, 16, 24)


def _rotl(x, r):
    return lax.shift_left(x, np.uint32(r)) | lax.shift_right_logical(
        x, np.uint32(32 - r)
    )


def _threefry_bits(c1):
    x0 = jnp.zeros_like(c1)
    x1 = c1 + _K1
    inject = ((_K1, _K2), (_K2, _K0), (_K0, _K1), (_K1, _K2), (_K2, _K0))
    for g in range(5):
        rots = _ROT[:4] if g % 2 == 0 else _ROT[4:]
        for r in rots:
            x0 = x0 + x1
            x1 = _rotl(x1, r)
            x1 = x1 ^ x0
        a, b = inject[g]
        x0 = x0 + a
        x1 = x1 + b + np.uint32(g + 1)
    return x0 ^ x1


def _scores(xc, flat, row_base_u32, t, mterm):
    c1 = flat.astype(jnp.uint32) + row_base_u32
    bits = _threefry_bits(c1)
    mant = lax.shift_right_logical(bits, np.uint32(9)) | np.uint32(0x3F800000)
    u = lax.bitcast_convert_type(mant, jnp.float32) - np.float32(1.0)
    expo = -jnp.log1p(-u)
    return (xc / t - mterm) - jnp.log(expo + EPS)


def _row_kernel(temps_ref, logits_ref, out_ref):
    r = pl.program_id(0)
    t_raw = temps_ref[r]
    t = jnp.where(t_raw == 0.0, np.float32(1.0), t_raw)
    row_base = r.astype(jnp.uint32) * np.uint32(COLS)

    x = logits_ref[...]
    col = lax.broadcasted_iota(jnp.int32, (SUB, LANES), 1)
    row8 = lax.broadcasted_iota(jnp.int32, (SUB, LANES), 0)
    flat_all = row8 * LANES + col

    vmax = jnp.max(x)
    greedy = jnp.min(jnp.where(x == vmax, flat_all, BIG))
    mterm = vmax / t

    colw = lax.broadcasted_iota(jnp.int32, (SUB, W), 1)
    roww = lax.broadcasted_iota(jnp.int32, (SUB, W), 0) * LANES

    def body(j, carry):
        m_s, idx = carry
        base = pl.multiple_of(j * W, W)
        xc = logits_ref[:, pl.ds(base, W)]
        flat = roww + base + colw
        s = _scores(xc, flat, row_base, t, mterm)
        upd = s > m_s
        return jnp.where(upd, s, m_s), jnp.where(upd, flat, idx)

    m0 = jnp.full((SUB, W), NEG_INF, jnp.float32)
    i0 = jnp.zeros((SUB, W), jnp.int32)
    m_s, idx = lax.fori_loop(0, NCHUNK, body, (m0, i0))
    smax_main = jnp.max(m_s)
    idx_main = jnp.min(jnp.where(m_s == smax_main, idx, BIG))

    xt = logits_ref[:, pl.ds(NCHUNK * W, TAIL)]
    flat_t = (
        lax.broadcasted_iota(jnp.int32, (SUB, TAIL), 0) * LANES
        + NCHUNK * W
        + lax.broadcasted_iota(jnp.int32, (SUB, TAIL), 1)
    )
    s_t = _scores(xt, flat_t, row_base, t, mterm)
    smax_t = jnp.max(s_t)
    idx_t = jnp.min(jnp.where(s_t == smax_t, flat_t, BIG))

    sample = jnp.where(
        smax_t > smax_main,
        idx_t,
        jnp.where(smax_t == smax_main, jnp.minimum(idx_t, idx_main), idx_main),
    )
    out_ref[r] = jnp.where(t_raw == 0.0, greedy, sample)


@jax.jit
def kernel(logits, temperatures):
    x = logits.reshape(ROWS * SUB, LANES)
    return pl.pallas_call(
        _row_kernel,
        grid=(ROWS,),
        in_specs=[
            pl.BlockSpec(memory_space=pltpu.SMEM),
            pl.BlockSpec((SUB, LANES), lambda r: (r, 0)),
        ],
        out_specs=pl.BlockSpec(memory_space=pltpu.SMEM),
        out_shape=jax.ShapeDtypeStruct((ROWS,), jnp.int32),
    )(temperatures, x)


# speedup vs baseline: 1.4512x; 1.0305x over previous
import numpy as np

import jax
import jax.numpy as jnp
from jax import lax
from jax.experimental import pallas as pl
from jax.experimental.pallas import tpu as pltpu

ROWS = 64
COLS = 1_000_000
SUB = 8
LANES = COLS // SUB
W = 1024
NCHUNK = LANES // W
TAIL = LANES - NCHUNK * W
EPS = np.float32(1e-10)
BIG = np.int32(2**30)
NEG_INF = np.float32("-inf")

_K0 = np.uint32(0)
_K1 = np.uint32(42)
_K2 = np.uint32(0x1BD11BDA ^ 42)
_ROT = (13, 15, 26, 6, 17, 29, 16, 24)


def _rotl(x, r):
    return lax.shift_left(x, np.uint32(r)) | lax.shift_right_logical(
        x, np.uint32(32 - r)
    )


def _threefry_bits(c1):
    x0 = jnp.zeros_like(c1)
    x1 = c1 + _K1
    inject = ((_K1, _K2), (_K2, _K0), (_K0, _K1), (_K1, _K2), (_K2, _K0))
    for g in range(5):
        rots = _ROT[:4] if g % 2 == 0 else _ROT[4:]
        for r in rots:
            x0 = x0 + x1
            x1 = _rotl(x1, r)
            x1 = x1 ^ x0
        a, b = inject[g]
        x0 = x0 + a
        x1 = x1 + b + np.uint32(g + 1)
    return x0 ^ x1


def _scores(xc, flat, row_base_u32, t, mterm):
    c1 = flat.astype(jnp.uint32) + row_base_u32
    bits = _threefry_bits(c1)
    mant = lax.shift_right_logical(bits, np.uint32(9)) | np.uint32(0x3F800000)
    u = lax.bitcast_convert_type(mant, jnp.float32) - np.float32(1.0)
    expo = -jnp.log1p(-u)
    return (xc / t - mterm) - jnp.log(expo + EPS)


def _row_kernel(temps_ref, logits_ref, out_ref):
    r = pl.program_id(0)
    t_raw = temps_ref[r]
    t = jnp.where(t_raw == 0.0, np.float32(1.0), t_raw)
    row_base = r.astype(jnp.uint32) * np.uint32(COLS)

    colw = lax.broadcasted_iota(jnp.int32, (SUB, W), 1)
    roww = lax.broadcasted_iota(jnp.int32, (SUB, W), 0) * LANES
    colt = lax.broadcasted_iota(jnp.int32, (SUB, TAIL), 1)
    rowt = lax.broadcasted_iota(jnp.int32, (SUB, TAIL), 0) * LANES
    flat_t = rowt + NCHUNK * W + colt

    def gbody(j, carry):
        gm, gidx = carry
        base = pl.multiple_of(j * W, W)
        xc = logits_ref[:, pl.ds(base, W)]
        upd = xc > gm
        return jnp.where(upd, xc, gm), jnp.where(upd, roww + base + colw, gidx)

    g0 = jnp.full((SUB, W), NEG_INF, jnp.float32)
    gm, gidx = lax.fori_loop(0, NCHUNK, gbody, (g0, jnp.zeros((SUB, W), jnp.int32)))
    xt = logits_ref[:, pl.ds(NCHUNK * W, TAIL)]
    vmax_main = jnp.max(gm)
    vmax_t = jnp.max(xt)
    vmax = jnp.maximum(vmax_main, vmax_t)
    g_main = jnp.min(jnp.where(gm == vmax, gidx, BIG))
    g_tail = jnp.min(jnp.where(xt == vmax, flat_t, BIG))
    greedy = jnp.minimum(g_main, g_tail)
    mterm = vmax / t

    def body(j, carry):
        m_s, idx = carry
        base = pl.multiple_of(j * W, W)
        xc = logits_ref[:, pl.ds(base, W)]
        flat = roww + base + colw
        s = _scores(xc, flat, row_base, t, mterm)
        upd = s > m_s
        return jnp.where(upd, s, m_s), jnp.where(upd, flat, idx)

    m0 = jnp.full((SUB, W), NEG_INF, jnp.float32)
    i0 = jnp.zeros((SUB, W), jnp.int32)
    m_s, idx = lax.fori_loop(0, NCHUNK, body, (m0, i0))
    smax_main = jnp.max(m_s)
    idx_main = jnp.min(jnp.where(m_s == smax_main, idx, BIG))

    s_t = _scores(xt, flat_t, row_base, t, mterm)
    smax_t = jnp.max(s_t)
    idx_t = jnp.min(jnp.where(s_t == smax_t, flat_t, BIG))

    sample = jnp.where(
        smax_t > smax_main,
        idx_t,
        jnp.where(smax_t == smax_main, jnp.minimum(idx_t, idx_main), idx_main),
    )
    out_ref[r] = jnp.where(t_raw == 0.0, greedy, sample)


@jax.jit
def kernel(logits, temperatures):
    x = logits.reshape(ROWS * SUB, LANES)
    return pl.pallas_call(
        _row_kernel,
        grid=(ROWS,),
        in_specs=[
            pl.BlockSpec(memory_space=pltpu.SMEM),
            pl.BlockSpec((SUB, LANES), lambda r: (r, 0)),
        ],
        out_specs=pl.BlockSpec(memory_space=pltpu.SMEM),
        out_shape=jax.ShapeDtypeStruct((ROWS,), jnp.int32),
    )(temperatures, x)


# speedup vs baseline: 1.5004x; 1.0339x over previous
import numpy as np

import jax
import jax.numpy as jnp
from jax import lax
from jax.experimental import pallas as pl
from jax.experimental.pallas import tpu as pltpu

ROWS = 64
COLS = 1_000_000
SUB = 8
LANES = COLS // SUB
W = 2048
NCHUNK = LANES // W
TAIL = LANES - NCHUNK * W
EPS = np.float32(1e-10)
BIG = np.int32(2**30)
NEG_INF = np.float32("-inf")

_K0 = np.uint32(0)
_K1 = np.uint32(42)
_K2 = np.uint32(0x1BD11BDA ^ 42)
_ROT = (13, 15, 26, 6, 17, 29, 16, 24)


def _rotl(x, r):
    return lax.shift_left(x, np.uint32(r)) | lax.shift_right_logical(
        x, np.uint32(32 - r)
    )


def _threefry_bits(x1):
    x0 = jnp.zeros_like(x1)
    inject = ((_K1, _K2), (_K2, _K0), (_K0, _K1), (_K1, _K2), (_K2, _K0))
    for g in range(5):
        rots = _ROT[:4] if g % 2 == 0 else _ROT[4:]
        for r in rots:
            x0 = x0 + x1
            x1 = _rotl(x1, r)
            x1 = x1 ^ x0
        a, b = inject[g]
        x0 = x0 + a
        x1 = x1 + np.uint32(b + g + 1)
    return x0 ^ x1


def _scores(xc, seed0, t, vmax):
    bits = _threefry_bits(seed0)
    mant = lax.shift_right_logical(bits, np.uint32(9)) | np.uint32(0x3F800000)
    f = lax.bitcast_convert_type(mant, jnp.float32)
    neg_u = np.float32(1.0) - f
    den = EPS - jnp.log1p(neg_u)
    return (xc - vmax) - t * jnp.log(den)


def _row_kernel(temps_ref, logits_ref, out_ref):
    r = pl.program_id(0)
    t_raw = temps_ref[r]
    t = jnp.where(t_raw == 0.0, np.float32(1.0), t_raw)
    row_base = r.astype(jnp.uint32) * np.uint32(COLS)

    colw = lax.broadcasted_iota(jnp.int32, (SUB, W), 1)
    roww = lax.broadcasted_iota(jnp.int32, (SUB, W), 0) * LANES
    w0 = roww + colw
    w0u = w0.astype(jnp.uint32)
    colt = lax.broadcasted_iota(jnp.int32, (SUB, TAIL), 1)
    rowt = lax.broadcasted_iota(jnp.int32, (SUB, TAIL), 0) * LANES
    flat_t = rowt + NCHUNK * W + colt

    def gbody(j, carry):
        gm, gj = carry
        base = pl.multiple_of(j * W, W)
        xc = logits_ref[:, pl.ds(base, W)]
        upd = xc > gm
        return jnp.where(upd, xc, gm), jnp.where(upd, j, gj)

    g0 = jnp.full((SUB, W), NEG_INF, jnp.float32)
    gm, gj = lax.fori_loop(0, NCHUNK, gbody, (g0, jnp.zeros((SUB, W), jnp.int32)))
    xt = logits_ref[:, pl.ds(NCHUNK * W, TAIL)]
    vmax = jnp.maximum(jnp.max(gm), jnp.max(xt))
    g_main = jnp.min(jnp.where(gm == vmax, w0 + gj * W, BIG))
    g_tail = jnp.min(jnp.where(xt == vmax, flat_t, BIG))
    greedy = jnp.minimum(g_main, g_tail)

    def body(j, carry):
        m_s, js = carry
        base = pl.multiple_of(j * W, W)
        xc = logits_ref[:, pl.ds(base, W)]
        seed0 = w0u + (base.astype(jnp.uint32) + row_base + np.uint32(42))
        s = _scores(xc, seed0, t, vmax)
        upd = s > m_s
        return jnp.where(upd, s, m_s), jnp.where(upd, j, js)

    m0 = jnp.full((SUB, W), NEG_INF, jnp.float32)
    m_s, js = lax.fori_loop(0, NCHUNK, body, (m0, jnp.zeros((SUB, W), jnp.int32)))
    smax_main = jnp.max(m_s)
    idx_main = jnp.min(jnp.where(m_s == smax_main, w0 + js * W, BIG))

    seed_t = flat_t.astype(jnp.uint32) + (row_base + np.uint32(42))
    s_t = _scores(xt, seed_t, t, vmax)
    smax_t = jnp.max(s_t)
    idx_t = jnp.min(jnp.where(s_t == smax_t, flat_t, BIG))

    sample = jnp.where(
        smax_t > smax_main,
        idx_t,
        jnp.where(smax_t == smax_main, jnp.minimum(idx_t, idx_main), idx_main),
    )
    out_ref[r] = jnp.where(t_raw == 0.0, greedy, sample)


@jax.jit
def kernel(logits, temperatures):
    x = logits.reshape(ROWS * SUB, LANES)
    return pl.pallas_call(
        _row_kernel,
        grid=(ROWS,),
        in_specs=[
            pl.BlockSpec(memory_space=pltpu.SMEM),
            pl.BlockSpec((SUB, LANES), lambda r: (r, 0)),
        ],
        out_specs=pl.BlockSpec(memory_space=pltpu.SMEM),
        out_shape=jax.ShapeDtypeStruct((ROWS,), jnp.int32),
    )(temperatures, x)
